# speedup vs baseline: 1.0227x; 1.0227x over previous
import jax
import jax.numpy as jnp
from jax import lax
from jax.experimental import pallas as pl
from jax.experimental.pallas import tpu as pltpu
from jax.experimental.pallas import tpu_sc as plsc

VOCAB = 100000
D = 1024
SCALE = 32.0

NC = 2
NS = 16
NW = NC * NS

B = 4 * 8192
B_PER_W = B // NW
C = 64
N_CHUNKS = B_PER_W // C


def _body(w_hbm, xi_hbm, out_hbm, idx_v, rows_v, sem):
    wid = lax.axis_index("s") * NC + lax.axis_index("c")
    base = wid * B_PER_W
    pltpu.sync_copy(xi_hbm.at[pl.ds(base, B_PER_W)], idx_v)

    def chunk_body(ci, carry):
        pltpu.async_copy(
            w_hbm.at[idx_v.at[pl.ds(ci * C, C)]], rows_v, sem
        ).wait()

        def row_body(i, c2):
            for j in range(D // 16):
                sl = (i, pl.ds(j * 16, 16))
                rows_v[sl] = rows_v[sl] * SCALE
            return c2

        lax.fori_loop(0, C, row_body, 0)
        pltpu.sync_copy(rows_v, out_hbm.at[pl.ds(base + ci * C, C)])
        return carry

    lax.fori_loop(0, N_CHUNKS, chunk_body, 0)


@jax.jit
def kernel(x, W):
    xflat = x.reshape(-1)
    mesh = plsc.VectorSubcoreMesh(
        core_axis_name="c", subcore_axis_name="s", num_cores=NC, num_subcores=NS
    )
    out = pl.kernel(
        _body,
        out_type=jax.ShapeDtypeStruct((B, D), jnp.float32),
        mesh=mesh,
        scratch_types=[
            pltpu.VMEM((B_PER_W,), jnp.int32),
            pltpu.VMEM((C, D), jnp.float32),
            pltpu.SemaphoreType.DMA,
        ],
    )(W, xflat)
    return out.reshape(x.shape[0], x.shape[1], D)


# speedup vs baseline: 1.6215x; 1.5855x over previous
import jax
import jax.numpy as jnp
from jax import lax
from jax.experimental import pallas as pl
from jax.experimental.pallas import tpu as pltpu
from jax.experimental.pallas import tpu_sc as plsc

D = 1024
SCALE = 32.0

NC = 2
NS = 16
NW = NC * NS

B = 4 * 8192
B_PER_W = B // NW
C = 16
N_CHUNKS = B_PER_W // C
N_PAIRS = N_CHUNKS // 2


def _body(w_hbm, xi_hbm, out_hbm, idx_v, gbuf, sbuf,
          gsem0, gsem1, ssem0, ssem1):
    wid = lax.axis_index("s") * NC + lax.axis_index("c")
    base = wid * B_PER_W
    pltpu.sync_copy(xi_hbm.at[pl.ds(base, B_PER_W)], idx_v)

    gsems = (gsem0, gsem1)
    ssems = (ssem0, ssem1)

    def gslot(b):
        return gbuf.at[pl.ds(b * C, C)]

    def sslot(b):
        return sbuf.at[pl.ds(b * C, C)]

    def issue_gather(ci, b):
        pltpu.async_copy(w_hbm.at[idx_v.at[pl.ds(ci * C, C)]],
                         gslot(b), gsems[b])

    issue_gather(0, 0)
    issue_gather(1, 1)

    def pair_body(k, carry):
        for b in range(2):
            ci = k * 2 + b
            pltpu.make_async_copy(w_hbm.at[pl.ds(0, C)], gslot(b),
                                  gsems[b]).wait()
            @pl.when(k > 0)
            def _():
                pltpu.make_async_copy(sslot(b), out_hbm.at[pl.ds(0, C)],
                                      ssems[b]).wait()

            def row_body(i, c2):
                for j in range(D // 16):
                    sl = (i, pl.ds(j * 16, 16))
                    sslot(b)[sl] = gslot(b)[sl] * SCALE
                return c2

            lax.fori_loop(0, C, row_body, 0)

            @pl.when(k < N_PAIRS - 1)
            def _():
                issue_gather(ci + 2, b)

            pltpu.async_copy(sslot(b), out_hbm.at[pl.ds(base + ci * C, C)],
                             ssems[b])
        return carry

    lax.fori_loop(0, N_PAIRS, pair_body, 0)

    for b in range(2):
        pltpu.make_async_copy(sslot(b), out_hbm.at[pl.ds(0, C)],
                              ssems[b]).wait()


@jax.jit
def kernel(x, W):
    xflat = x.reshape(-1)
    mesh = plsc.VectorSubcoreMesh(
        core_axis_name="c", subcore_axis_name="s", num_cores=NC, num_subcores=NS
    )
    out = pl.kernel(
        _body,
        out_type=jax.ShapeDtypeStruct((B, D), jnp.float32),
        mesh=mesh,
        scratch_types=[
            pltpu.VMEM((B_PER_W,), jnp.int32),
            pltpu.VMEM((2 * C, D), jnp.float32),
            pltpu.VMEM((2 * C, D), jnp.float32),
            pltpu.SemaphoreType.DMA,
            pltpu.SemaphoreType.DMA,
            pltpu.SemaphoreType.DMA,
            pltpu.SemaphoreType.DMA,
        ],
    )(W, xflat)
    return out.reshape(x.shape[0], x.shape[1], D)
